# baseline (speedup 1.0000x reference)
import functools

import jax
import jax.numpy as jnp
from jax import lax
from jax.experimental import pallas as pl
from jax.experimental.pallas import tpu as pltpu

_N = 5000
_K = 300
_SIGMA = 0.5
_BLK = 512
_NPAD = 5120
_NBLK = _NPAD // _BLK
_NEG = -1e30


def _rank_body(sr, sc, rank_ref):
    def outer(i, _):
        s_r = sr[pl.ds(i * _BLK, _BLK), :]

        def lower(j, acc):
            s_c = sc[:, pl.ds(j * _BLK, _BLK)]
            m = (s_c >= s_r).astype(jnp.float32)
            return acc + jnp.sum(m, axis=1, keepdims=True)

        def upper(j, acc):
            s_c = sc[:, pl.ds(j * _BLK, _BLK)]
            m = (s_c > s_r).astype(jnp.float32)
            return acc + jnp.sum(m, axis=1, keepdims=True)

        acc = lax.fori_loop(0, i, lower, jnp.zeros((_BLK, 1), jnp.float32))
        acc = lax.fori_loop(i + 1, _NBLK, upper, acc)
        s_c = sc[:, pl.ds(i * _BLK, _BLK)]
        ridx = lax.broadcasted_iota(jnp.int32, (_BLK, _BLK), 0)
        cidx = lax.broadcasted_iota(jnp.int32, (_BLK, _BLK), 1)
        m = ((s_c > s_r) | ((s_c == s_r) & (cidx < ridx))).astype(jnp.float32)
        acc = acc + jnp.sum(m, axis=1, keepdims=True)
        rank_ref[pl.ds(i * _BLK, _BLK), :] = acc.astype(jnp.int32)
        return 0

    lax.fori_loop(0, _NBLK, outer, 0)


def _sweep_body(x1r, y1r, x2r, y2r, ar, sr, x1c, y1c, x2c, y2c, ac,
                out_ref, c2c_ref, cpref_ref, qpref_ref):
    rrefs = (x1r, y1r, x2r, y2r, ar)
    crefs = (x1c, y1c, x2c, y2c, ac)

    def _iou(rows, j):
        rx1, ry1, rx2, ry2, ra = rows
        cx1, cy1, cx2, cy2, ca = [r[:, pl.ds(j * _BLK, _BLK)] for r in crefs]
        iw = jnp.maximum(jnp.minimum(rx2, cx2) - jnp.maximum(rx1, cx1), 0.0)
        ih = jnp.maximum(jnp.minimum(ry2, cy2) - jnp.maximum(ry1, cy1), 0.0)
        inter = iw * ih
        return inter / ((ra + ca) - inter)

    def outer(i, _):
        def strip(t, _):
            g0 = i * _BLK + t * 8
            rowp = [r[pl.ds(g0, 8), :] for r in rrefs]

            def inner(j, carry):
                acc_c, acc_q = carry
                iou = _iou(rowp, j)
                acc_c = jnp.maximum(acc_c, iou)
                acc_q = jnp.maximum(
                    acc_q, iou * iou - c2c_ref[:, pl.ds(j * _BLK, _BLK)])
                return acc_c, acc_q

            acc_c, acc_q = lax.fori_loop(
                0, i, inner,
                (jnp.zeros((8, _BLK), jnp.float32),
                 jnp.full((8, _BLK), _NEG, jnp.float32)))
            cpref_ref[pl.ds(t * 8, 8), :] = jnp.max(acc_c, axis=1,
                                                    keepdims=True)
            qpref_ref[pl.ds(t * 8, 8), :] = jnp.max(acc_q, axis=1,
                                                    keepdims=True)
            return 0

        lax.fori_loop(0, _BLK // 8, strip, 0)
        rows = [r[pl.ds(i * _BLK, _BLK), :] for r in rrefs]
        acc_c = cpref_ref[...]
        acc_q = qpref_ref[...]

        iou_d = _iou(rows, i)
        ridx = lax.broadcasted_iota(jnp.int32, (_BLK, _BLK), 0)
        cidx = lax.broadcasted_iota(jnp.int32, (_BLK, _BLK), 1)
        mlow = cidx < ridx
        comp_r = jnp.maximum(
            acc_c,
            jnp.max(jnp.where(mlow, iou_d, 0.0), axis=1, keepdims=True))
        comp2_r = comp_r * comp_r
        comp2_c = jnp.max(jnp.where(ridx == cidx, comp2_r, 0.0), axis=0,
                          keepdims=True)
        c2c_ref[:, pl.ds(i * _BLK, _BLK)] = comp2_c
        qd = jnp.where(mlow, iou_d * iou_d - comp2_c, _NEG)
        q = jnp.maximum(acc_q, jnp.max(qd, axis=1, keepdims=True))

        s_r = sr[pl.ds(i * _BLK, _BLK), :]
        new_s = s_r * jnp.exp(-jnp.maximum(q, 0.0) / _SIGMA)
        gidx = i * _BLK + lax.broadcasted_iota(jnp.int32, (_BLK, 1), 0)
        out_ref[pl.ds(i * _BLK, _BLK), :] = jnp.where(gidx < _N, new_s, _NEG)
        return 0

    lax.fori_loop(0, _NBLK, outer, 0)


@jax.jit
def kernel(boxes, scores):
    pad = _NPAD - _N
    b = jnp.pad(boxes, ((0, pad), (0, 0)))
    s = jnp.pad(scores, (0, pad), constant_values=-1.0)

    x1 = jnp.minimum(b[:, 0], b[:, 2])
    y1 = jnp.minimum(b[:, 1], b[:, 3])
    x2 = jnp.maximum(b[:, 0], b[:, 2])
    y2 = jnp.maximum(b[:, 1], b[:, 3])
    area = (x2 - x1) * (y2 - y1) + 1e-8

    f32 = jnp.float32
    rank = pl.pallas_call(
        _rank_body,
        out_shape=jax.ShapeDtypeStruct((_NPAD, 1), jnp.int32),
    )(s[:, None], s[None, :])[:, 0]

    packed = jnp.stack([b[:, 0], b[:, 1], b[:, 2], b[:, 3],
                        x1, y1, x2, y2, area, s], axis=1)
    sortedp = jnp.zeros((_NPAD, 10), f32).at[rank].set(packed)
    sortedpt = sortedp.T

    row = lambda k: sortedp[:, k:k + 1]
    col = lambda k: sortedpt[k:k + 1, :]
    new_s = pl.pallas_call(
        _sweep_body,
        out_shape=(jax.ShapeDtypeStruct((_NPAD, 1), f32),
                   jax.ShapeDtypeStruct((1, _NPAD), f32)),
        scratch_shapes=[pltpu.VMEM((_BLK, 1), f32),
                        pltpu.VMEM((_BLK, 1), f32)],
    )(row(4), row(5), row(6), row(7), row(8), row(9),
      col(4), col(5), col(6), col(7), col(8))[0]

    vals, idx = lax.top_k(new_s[:, 0], _K)
    sel = jnp.take(sortedp[:, 0:4], idx, axis=0)
    return jnp.concatenate([sel, vals[:, None]], axis=1)


# speedup vs baseline: 3.2934x; 3.2934x over previous
import functools

import jax
import jax.numpy as jnp
from jax import lax
from jax.experimental import pallas as pl

_N = 5000
_K = 300
_SIGMA = 0.5
_BLK = 512
_NPAD = 5120
_NBLK = _NPAD // _BLK
_NEG = -1e30


def _rank_body(sr, sc, rank_ref):
    def outer(i, _):
        s_r = sr[pl.ds(i * _BLK, _BLK), :]

        def lower(j, acc):
            s_c = sc[:, pl.ds(j * _BLK, _BLK)]
            m = (s_c >= s_r).astype(jnp.float32)
            return acc + jnp.sum(m, axis=1, keepdims=True)

        def upper(j, acc):
            s_c = sc[:, pl.ds(j * _BLK, _BLK)]
            m = (s_c > s_r).astype(jnp.float32)
            return acc + jnp.sum(m, axis=1, keepdims=True)

        acc = lax.fori_loop(0, i, lower, jnp.zeros((_BLK, 1), jnp.float32))
        acc = lax.fori_loop(i + 1, _NBLK, upper, acc)
        s_c = sc[:, pl.ds(i * _BLK, _BLK)]
        ridx = lax.broadcasted_iota(jnp.int32, (_BLK, _BLK), 0)
        cidx = lax.broadcasted_iota(jnp.int32, (_BLK, _BLK), 1)
        m = ((s_c > s_r) | ((s_c == s_r) & (cidx < ridx))).astype(jnp.float32)
        acc = acc + jnp.sum(m, axis=1, keepdims=True)
        rank_ref[pl.ds(i * _BLK, _BLK), :] = acc.astype(jnp.int32)
        return 0

    lax.fori_loop(0, _NBLK, outer, 0)


def _sweep_body(x1r, y1r, x2r, y2r, ar, sr, x1c, y1c, x2c, y2c, ac,
                out_ref, c2c_ref):
    rrefs = (x1r, y1r, x2r, y2r, ar)
    crefs = (x1c, y1c, x2c, y2c, ac)

    def _iou(rows, j):
        rx1, ry1, rx2, ry2, ra = rows
        cx1, cy1, cx2, cy2, ca = [r[:, pl.ds(j * _BLK, _BLK)] for r in crefs]
        iw = jnp.maximum(jnp.minimum(rx2, cx2) - jnp.maximum(rx1, cx1), 0.0)
        ih = jnp.maximum(jnp.minimum(ry2, cy2) - jnp.maximum(ry1, cy1), 0.0)
        inter = iw * ih
        return inter / ((ra + ca) - inter)

    def outer(i, _):
        rows = [r[pl.ds(i * _BLK, _BLK), :] for r in rrefs]

        def inner(j, carry):
            acc_c, acc_q = carry
            iou = _iou(rows, j)
            acc_c = jnp.maximum(acc_c, jnp.max(iou, axis=1, keepdims=True))
            val = iou * iou - c2c_ref[:, pl.ds(j * _BLK, _BLK)]
            acc_q = jnp.maximum(acc_q, jnp.max(val, axis=1, keepdims=True))
            return acc_c, acc_q

        init = (jnp.zeros((_BLK, 1), jnp.float32),
                jnp.full((_BLK, 1), _NEG, jnp.float32))
        if i == 0:
            acc_c, acc_q = init
        else:
            acc_c, acc_q = lax.fori_loop(0, i, inner, init,
                                         unroll=min(i, 2))

        iou_d = _iou(rows, i)
        ridx = lax.broadcasted_iota(jnp.int32, (_BLK, _BLK), 0)
        cidx = lax.broadcasted_iota(jnp.int32, (_BLK, _BLK), 1)
        mlow = cidx < ridx
        comp_r = jnp.maximum(
            acc_c,
            jnp.max(jnp.where(mlow, iou_d, 0.0), axis=1, keepdims=True))
        comp2_r = comp_r * comp_r
        comp2_c = jnp.max(jnp.where(ridx == cidx, comp2_r, 0.0), axis=0,
                          keepdims=True)
        c2c_ref[:, pl.ds(i * _BLK, _BLK)] = comp2_c
        qd = jnp.where(mlow, iou_d * iou_d - comp2_c, _NEG)
        q = jnp.maximum(acc_q, jnp.max(qd, axis=1, keepdims=True))

        s_r = sr[pl.ds(i * _BLK, _BLK), :]
        new_s = s_r * jnp.exp(-jnp.maximum(q, 0.0) / _SIGMA)
        gidx = i * _BLK + lax.broadcasted_iota(jnp.int32, (_BLK, 1), 0)
        out_ref[pl.ds(i * _BLK, _BLK), :] = jnp.where(gidx < _N, new_s, _NEG)

    for i in range(_NBLK):
        outer(i, None)


@jax.jit
def kernel(boxes, scores):
    pad = _NPAD - _N
    b = jnp.pad(boxes, ((0, pad), (0, 0)))
    s = jnp.pad(scores, (0, pad), constant_values=-1.0)

    x1 = jnp.minimum(b[:, 0], b[:, 2])
    y1 = jnp.minimum(b[:, 1], b[:, 3])
    x2 = jnp.maximum(b[:, 0], b[:, 2])
    y2 = jnp.maximum(b[:, 1], b[:, 3])
    area = (x2 - x1) * (y2 - y1) + 1e-8

    f32 = jnp.float32
    rank = pl.pallas_call(
        _rank_body,
        out_shape=jax.ShapeDtypeStruct((_NPAD, 1), jnp.int32),
    )(s[:, None], s[None, :])[:, 0]

    packed = jnp.stack([b[:, 0], b[:, 1], b[:, 2], b[:, 3],
                        x1, y1, x2, y2, area, s], axis=1)
    sortedp = jnp.zeros((_NPAD, 10), f32).at[rank].set(packed)
    sortedpt = sortedp.T

    row = lambda k: sortedp[:, k:k + 1]
    col = lambda k: sortedpt[k:k + 1, :]
    new_s = pl.pallas_call(
        _sweep_body,
        out_shape=(jax.ShapeDtypeStruct((_NPAD, 1), f32),
                   jax.ShapeDtypeStruct((1, _NPAD), f32)),
    )(row(4), row(5), row(6), row(7), row(8), row(9),
      col(4), col(5), col(6), col(7), col(8))[0]

    vals, idx = lax.top_k(new_s[:, 0], _K)
    sel = jnp.take(sortedp[:, 0:4], idx, axis=0)
    return jnp.concatenate([sel, vals[:, None]], axis=1)


# speedup vs baseline: 3.5805x; 1.0872x over previous
import functools

import jax
import jax.numpy as jnp
from jax import lax
from jax.experimental import pallas as pl

_N = 5000
_K = 300
_SIGMA = 0.5
_BLK = 512
_NPAD = 5120
_NBLK = _NPAD // _BLK
_NEG = -1e30


def _rank_body(sr, sc, rank_ref):
    def outer(i, _):
        s_r = sr[pl.ds(i * _BLK, _BLK), :]

        def lower(j, acc):
            s_c = sc[:, pl.ds(j * _BLK, _BLK)]
            m = (s_c >= s_r).astype(jnp.float32)
            return acc + jnp.sum(m, axis=1, keepdims=True)

        def upper(j, acc):
            s_c = sc[:, pl.ds(j * _BLK, _BLK)]
            m = (s_c > s_r).astype(jnp.float32)
            return acc + jnp.sum(m, axis=1, keepdims=True)

        acc = jnp.zeros((_BLK, 1), jnp.float32)
        if i > 0:
            acc = lax.fori_loop(0, i, lower, acc, unroll=min(i, 4))
        if i + 1 < _NBLK:
            acc = lax.fori_loop(i + 1, _NBLK, upper, acc,
                                unroll=min(_NBLK - 1 - i, 4))
        s_c = sc[:, pl.ds(i * _BLK, _BLK)]
        ridx = lax.broadcasted_iota(jnp.int32, (_BLK, _BLK), 0)
        cidx = lax.broadcasted_iota(jnp.int32, (_BLK, _BLK), 1)
        m = ((s_c > s_r) | ((s_c == s_r) & (cidx < ridx))).astype(jnp.float32)
        acc = acc + jnp.sum(m, axis=1, keepdims=True)
        rank_ref[pl.ds(i * _BLK, _BLK), :] = acc.astype(jnp.int32)

    for i in range(_NBLK):
        outer(i, None)


def _sweep_body(x1r, y1r, x2r, y2r, ar, sr, x1c, y1c, x2c, y2c, ac,
                out_ref, c2c_ref):
    rrefs = (x1r, y1r, x2r, y2r, ar)
    crefs = (x1c, y1c, x2c, y2c, ac)

    def _iou(rows, j):
        rx1, ry1, rx2, ry2, ra = rows
        cx1, cy1, cx2, cy2, ca = [r[:, pl.ds(j * _BLK, _BLK)] for r in crefs]
        iw = jnp.maximum(jnp.minimum(rx2, cx2) - jnp.maximum(rx1, cx1), 0.0)
        ih = jnp.maximum(jnp.minimum(ry2, cy2) - jnp.maximum(ry1, cy1), 0.0)
        inter = iw * ih
        return inter / ((ra + ca) - inter)

    def outer(i, _):
        rows = [r[pl.ds(i * _BLK, _BLK), :] for r in rrefs]

        def inner(j, carry):
            acc_c, acc_q = carry
            iou = _iou(rows, j)
            acc_c = jnp.maximum(acc_c, jnp.max(iou, axis=1, keepdims=True))
            val = iou * iou - c2c_ref[:, pl.ds(j * _BLK, _BLK)]
            acc_q = jnp.maximum(acc_q, jnp.max(val, axis=1, keepdims=True))
            return acc_c, acc_q

        init = (jnp.zeros((_BLK, 1), jnp.float32),
                jnp.full((_BLK, 1), _NEG, jnp.float32))
        if i == 0:
            acc_c, acc_q = init
        else:
            acc_c, acc_q = lax.fori_loop(0, i, inner, init,
                                         unroll=min(i, 4))

        iou_d = _iou(rows, i)
        ridx = lax.broadcasted_iota(jnp.int32, (_BLK, _BLK), 0)
        cidx = lax.broadcasted_iota(jnp.int32, (_BLK, _BLK), 1)
        mlow = cidx < ridx
        comp_r = jnp.maximum(
            acc_c,
            jnp.max(jnp.where(mlow, iou_d, 0.0), axis=1, keepdims=True))
        comp2_r = comp_r * comp_r
        comp2_c = jnp.max(jnp.where(ridx == cidx, comp2_r, 0.0), axis=0,
                          keepdims=True)
        c2c_ref[:, pl.ds(i * _BLK, _BLK)] = comp2_c
        qd = jnp.where(mlow, iou_d * iou_d - comp2_c, _NEG)
        q = jnp.maximum(acc_q, jnp.max(qd, axis=1, keepdims=True))

        s_r = sr[pl.ds(i * _BLK, _BLK), :]
        new_s = s_r * jnp.exp(-jnp.maximum(q, 0.0) / _SIGMA)
        gidx = i * _BLK + lax.broadcasted_iota(jnp.int32, (_BLK, 1), 0)
        out_ref[pl.ds(i * _BLK, _BLK), :] = jnp.where(gidx < _N, new_s, _NEG)

    for i in range(_NBLK):
        outer(i, None)


@jax.jit
def kernel(boxes, scores):
    pad = _NPAD - _N
    b = jnp.pad(boxes, ((0, pad), (0, 0)))
    s = jnp.pad(scores, (0, pad), constant_values=-1.0)

    x1 = jnp.minimum(b[:, 0], b[:, 2])
    y1 = jnp.minimum(b[:, 1], b[:, 3])
    x2 = jnp.maximum(b[:, 0], b[:, 2])
    y2 = jnp.maximum(b[:, 1], b[:, 3])
    area = (x2 - x1) * (y2 - y1) + 1e-8

    f32 = jnp.float32
    rank = pl.pallas_call(
        _rank_body,
        out_shape=jax.ShapeDtypeStruct((_NPAD, 1), jnp.int32),
    )(s[:, None], s[None, :])[:, 0]

    packed = jnp.stack([b[:, 0], b[:, 1], b[:, 2], b[:, 3],
                        x1, y1, x2, y2, area, s], axis=1)
    sortedp = jnp.zeros((_NPAD, 10), f32).at[rank].set(packed)
    sortedpt = sortedp.T

    row = lambda k: sortedp[:, k:k + 1]
    col = lambda k: sortedpt[k:k + 1, :]
    new_s = pl.pallas_call(
        _sweep_body,
        out_shape=(jax.ShapeDtypeStruct((_NPAD, 1), f32),
                   jax.ShapeDtypeStruct((1, _NPAD), f32)),
    )(row(4), row(5), row(6), row(7), row(8), row(9),
      col(4), col(5), col(6), col(7), col(8))[0]

    vals, idx = lax.top_k(new_s[:, 0], _K)
    sel = jnp.take(sortedp[:, 0:4], idx, axis=0)
    return jnp.concatenate([sel, vals[:, None]], axis=1)


# speedup vs baseline: 4.0121x; 1.1205x over previous
import functools

import jax
import jax.numpy as jnp
from jax import lax
from jax.experimental import pallas as pl

_N = 5000
_K = 300
_SIGMA = 0.5
_BLK = 512
_NPAD = 5120
_NBLK = _NPAD // _BLK
_NEG = -1e30


def _rank_body(sr, sc, rank_ref):
    def outer(i, _):
        s_r = sr[pl.ds(i * _BLK, _BLK), :]

        def lower(j, acc):
            s_c = sc[:, pl.ds(j * _BLK, _BLK)]
            m = (s_c >= s_r).astype(jnp.float32)
            return acc + jnp.sum(m, axis=1, keepdims=True)

        def upper(j, acc):
            s_c = sc[:, pl.ds(j * _BLK, _BLK)]
            m = (s_c > s_r).astype(jnp.float32)
            return acc + jnp.sum(m, axis=1, keepdims=True)

        acc = jnp.zeros((_BLK, 1), jnp.float32)
        if i > 0:
            acc = lax.fori_loop(0, i, lower, acc, unroll=min(i, 8))
        if i + 1 < _NBLK:
            acc = lax.fori_loop(i + 1, _NBLK, upper, acc,
                                unroll=min(_NBLK - 1 - i, 8))
        s_c = sc[:, pl.ds(i * _BLK, _BLK)]
        ridx = lax.broadcasted_iota(jnp.int32, (_BLK, _BLK), 0)
        cidx = lax.broadcasted_iota(jnp.int32, (_BLK, _BLK), 1)
        m = ((s_c > s_r) | ((s_c == s_r) & (cidx < ridx))).astype(jnp.float32)
        acc = acc + jnp.sum(m, axis=1, keepdims=True)
        rank_ref[pl.ds(i * _BLK, _BLK), :] = acc.astype(jnp.int32)

    for i in range(_NBLK):
        outer(i, None)


def _sweep_body(x1r, y1r, x2r, y2r, ar, sr, x1c, y1c, x2c, y2c, ac,
                out_ref, c2c_ref):
    rrefs = (x1r, y1r, x2r, y2r, ar)
    crefs = (x1c, y1c, x2c, y2c, ac)

    def _iou(rows, j):
        rx1, ry1, rx2, ry2, ra = rows
        cx1, cy1, cx2, cy2, ca = [r[:, pl.ds(j * _BLK, _BLK)] for r in crefs]
        iw = jnp.maximum(jnp.minimum(rx2, cx2) - jnp.maximum(rx1, cx1), 0.0)
        ih = jnp.maximum(jnp.minimum(ry2, cy2) - jnp.maximum(ry1, cy1), 0.0)
        inter = iw * ih
        return inter / ((ra + ca) - inter)

    def outer(i, _):
        rows = [r[pl.ds(i * _BLK, _BLK), :] for r in rrefs]

        def inner(j, carry):
            acc_c, acc_q = carry
            iou = _iou(rows, j)
            acc_c = jnp.maximum(acc_c, jnp.max(iou, axis=1, keepdims=True))
            val = iou * iou - c2c_ref[:, pl.ds(j * _BLK, _BLK)]
            acc_q = jnp.maximum(acc_q, jnp.max(val, axis=1, keepdims=True))
            return acc_c, acc_q

        init = (jnp.zeros((_BLK, 1), jnp.float32),
                jnp.full((_BLK, 1), _NEG, jnp.float32))
        if i == 0:
            acc_c, acc_q = init
        else:
            acc_c, acc_q = lax.fori_loop(0, i, inner, init,
                                         unroll=min(i, 8))

        iou_d = _iou(rows, i)
        ridx = lax.broadcasted_iota(jnp.int32, (_BLK, _BLK), 0)
        cidx = lax.broadcasted_iota(jnp.int32, (_BLK, _BLK), 1)
        mlow = cidx < ridx
        comp_r = jnp.maximum(
            acc_c,
            jnp.max(jnp.where(mlow, iou_d, 0.0), axis=1, keepdims=True))
        comp2_r = comp_r * comp_r
        comp2_c = jnp.max(jnp.where(ridx == cidx, comp2_r, 0.0), axis=0,
                          keepdims=True)
        c2c_ref[:, pl.ds(i * _BLK, _BLK)] = comp2_c
        qd = jnp.where(mlow, iou_d * iou_d - comp2_c, _NEG)
        q = jnp.maximum(acc_q, jnp.max(qd, axis=1, keepdims=True))

        s_r = sr[pl.ds(i * _BLK, _BLK), :]
        new_s = s_r * jnp.exp(-jnp.maximum(q, 0.0) / _SIGMA)
        gidx = i * _BLK + lax.broadcasted_iota(jnp.int32, (_BLK, 1), 0)
        out_ref[pl.ds(i * _BLK, _BLK), :] = jnp.where(gidx < _N, new_s, _NEG)

    for i in range(_NBLK):
        outer(i, None)


@jax.jit
def kernel(boxes, scores):
    pad = _NPAD - _N
    b = jnp.pad(boxes, ((0, pad), (0, 0)))
    s = jnp.pad(scores, (0, pad), constant_values=-1.0)

    x1 = jnp.minimum(b[:, 0], b[:, 2])
    y1 = jnp.minimum(b[:, 1], b[:, 3])
    x2 = jnp.maximum(b[:, 0], b[:, 2])
    y2 = jnp.maximum(b[:, 1], b[:, 3])
    area = (x2 - x1) * (y2 - y1) + 1e-8

    f32 = jnp.float32
    rank = pl.pallas_call(
        _rank_body,
        out_shape=jax.ShapeDtypeStruct((_NPAD, 1), jnp.int32),
    )(s[:, None], s[None, :])[:, 0]

    packed = jnp.stack([b[:, 0], b[:, 1], b[:, 2], b[:, 3],
                        x1, y1, x2, y2, area, s], axis=1)
    sortedp = jnp.zeros((_NPAD, 10), f32).at[rank].set(packed)
    sortedpt = sortedp.T

    row = lambda k: sortedp[:, k:k + 1]
    col = lambda k: sortedpt[k:k + 1, :]
    new_s = pl.pallas_call(
        _sweep_body,
        out_shape=(jax.ShapeDtypeStruct((_NPAD, 1), f32),
                   jax.ShapeDtypeStruct((1, _NPAD), f32)),
    )(row(4), row(5), row(6), row(7), row(8), row(9),
      col(4), col(5), col(6), col(7), col(8))[0]

    vals, idx = lax.top_k(new_s[:, 0], _K)
    sel = jnp.take(sortedp[:, 0:4], idx, axis=0)
    return jnp.concatenate([sel, vals[:, None]], axis=1)
